# baseline (speedup 1.0000x reference)
import functools

import jax
import jax.numpy as jnp
from jax import lax
from jax.experimental import pallas as pl
from jax.experimental.pallas import tpu as pltpu
from jax.experimental.pallas import tpu_sc as plsc

D_EMBED = 64
SCALE = 8.0

_INFO = plsc.get_sparse_core_info()
_NC, _NS, _L = _INFO.num_cores, _INFO.num_subcores, _INFO.num_lanes
_NW = _NC * _NS

IDX_ROW = 128
K = 8


@functools.lru_cache(maxsize=None)
def _make_kernel(n_idx_rows: int):
    rows_per_w = n_idx_rows // _NW
    n_chunks = rows_per_w // K
    C = K * IDX_ROW

    mesh = plsc.VectorSubcoreMesh(core_axis_name="c", subcore_axis_name="s")

    @functools.partial(
        pl.kernel,
        mesh=mesh,
        out_type=jax.ShapeDtypeStruct((n_idx_rows * IDX_ROW, D_EMBED),
                                      jnp.float32),
        scratch_types=[
            pltpu.VMEM((K, IDX_ROW), jnp.int32),
            pltpu.VMEM((C, D_EMBED), jnp.float32),
            pltpu.SemaphoreType.DMA,
        ],
        compiler_params=pltpu.CompilerParams(use_tc_tiling_on_sc=False),
    )
    def emb_kernel(table_hbm, idx_hbm, out_hbm, idx_v, rows_v, sem):
        wid = lax.axis_index("s") * _NC + lax.axis_index("c")
        row0 = wid * rows_per_w

        def chunk_body(c, carry):
            rbase = row0 + c * K
            pltpu.sync_copy(idx_hbm.at[pl.ds(rbase, K)], idx_v)
            copies = [
                pltpu.async_copy(
                    table_hbm.at[idx_v.at[j]],
                    rows_v.at[pl.ds(j * IDX_ROW, IDX_ROW)],
                    sem,
                )
                for j in range(K)
            ]
            for cp in copies:
                cp.wait()

            def scale_body(i, carry2):
                for j in range(D_EMBED // _L):
                    sl = pl.ds(j * _L, _L)
                    rows_v[i, sl] = rows_v[i, sl] * SCALE
                return carry2

            lax.fori_loop(0, C, scale_body, 0, unroll=2)

            pltpu.sync_copy(rows_v, out_hbm.at[pl.ds(rbase * IDX_ROW, C)])
            return carry

        lax.fori_loop(0, n_chunks, chunk_body, 0)

    return emb_kernel


def kernel(x, emb_tab):
    b0, b1 = x.shape
    n = b0 * b1
    idx2 = x.astype(jnp.int32).reshape(n // IDX_ROW, IDX_ROW)
    out = _make_kernel(n // IDX_ROW)(emb_tab, idx2)
    return out.reshape(b0, b1, D_EMBED)


# speedup vs baseline: 1.0644x; 1.0644x over previous
import functools

import jax
import jax.numpy as jnp
from jax import lax
from jax.experimental import pallas as pl
from jax.experimental.pallas import tpu as pltpu
from jax.experimental.pallas import tpu_sc as plsc

D_EMBED = 64
SCALE = 8.0

_INFO = plsc.get_sparse_core_info()
_NC, _NS, _L = _INFO.num_cores, _INFO.num_subcores, _INFO.num_lanes
_NW = _NC * _NS

IDX_ROW = 128
K = 2
NBUF = 4
UNROLL = 4


@functools.lru_cache(maxsize=None)
def _make_kernel(n_idx_rows: int):
    rows_per_w = n_idx_rows // _NW
    n_groups = rows_per_w // K
    C = K * IDX_ROW

    mesh = plsc.VectorSubcoreMesh(core_axis_name="c", subcore_axis_name="s")

    scratch = (
        [pltpu.VMEM((rows_per_w, IDX_ROW), jnp.int32)]
        + [pltpu.VMEM((C, D_EMBED), jnp.float32) for _ in range(NBUF)]
        + [pltpu.SemaphoreType.DMA for _ in range(2 * NBUF)]
    )

    @functools.partial(
        pl.kernel,
        mesh=mesh,
        out_type=jax.ShapeDtypeStruct((n_idx_rows * IDX_ROW, D_EMBED),
                                      jnp.float32),
        scratch_types=scratch,
        compiler_params=pltpu.CompilerParams(use_tc_tiling_on_sc=False),
    )
    def emb_kernel(table_hbm, idx_hbm, out_hbm, idx_all, *scr):
        row_bufs = scr[:NBUF]
        gsems = scr[NBUF:2 * NBUF]
        osems = scr[2 * NBUF:3 * NBUF]

        wid = lax.axis_index("s") * _NC + lax.axis_index("c")
        row0 = wid * rows_per_w

        pltpu.sync_copy(idx_hbm.at[pl.ds(row0, rows_per_w)], idx_all)

        def start_group(g, b):
            for j in range(K):
                pltpu.async_copy(
                    table_hbm.at[idx_all.at[g * K + j]],
                    row_bufs[b].at[pl.ds(j * IDX_ROW, IDX_ROW)],
                    gsems[b],
                )

        def drain_gather(b):
            pltpu.make_async_copy(
                table_hbm.at[pl.ds(0, C)], row_bufs[b], gsems[b]).wait()

        def drain_out(b):
            pltpu.make_async_copy(
                row_bufs[b], out_hbm.at[pl.ds(0, C)], osems[b]).wait()

        def scale(b):
            rows_v = row_bufs[b]

            def body(i, carry):
                for u in range(UNROLL):
                    for j in range(D_EMBED // _L):
                        sl = pl.ds(j * _L, _L)
                        rows_v[i * UNROLL + u, sl] = (
                            rows_v[i * UNROLL + u, sl] * SCALE)
                return carry

            lax.fori_loop(0, C // UNROLL, body, 0)

        start_group(0, 0)

        def outer(i0, carry):
            for b in range(NBUF):
                g = i0 * NBUF + b
                nb = (b + 1) % NBUF
                nxt = g + 1

                @pl.when(nxt < n_groups)
                def _():
                    @pl.when(g >= NBUF - 1)
                    def _():
                        drain_out(nb)

                    start_group(nxt, nb)

                drain_gather(b)
                scale(b)
                pltpu.async_copy(
                    row_bufs[b],
                    out_hbm.at[pl.ds((row0 + g * K) * IDX_ROW, C)],
                    osems[b],
                )
            return carry

        lax.fori_loop(0, n_groups // NBUF, outer, 0)

        for g in range(n_groups - NBUF, n_groups):
            drain_out(g % NBUF)

    return emb_kernel


def kernel(x, emb_tab):
    b0, b1 = x.shape
    n = b0 * b1
    idx2 = x.astype(jnp.int32).reshape(n // IDX_ROW, IDX_ROW)
    out = _make_kernel(n // IDX_ROW)(emb_tab, idx2)
    return out.reshape(b0, b1, D_EMBED)


# speedup vs baseline: 1.0649x; 1.0004x over previous
import functools

import jax
import jax.numpy as jnp
from jax import lax
from jax.experimental import pallas as pl
from jax.experimental.pallas import tpu as pltpu
from jax.experimental.pallas import tpu_sc as plsc

D_EMBED = 64
SCALE = 8.0

_INFO = plsc.get_sparse_core_info()
_NC, _NS, _L = _INFO.num_cores, _INFO.num_subcores, _INFO.num_lanes
_NW = _NC * _NS

IDX_ROW = 128
K = 2
NBUF = 4
UNROLL = 4


@functools.lru_cache(maxsize=None)
def _make_kernel(n_idx_rows: int):
    rows_per_w = n_idx_rows // _NW
    n_groups = rows_per_w // K
    C = K * IDX_ROW

    mesh = plsc.VectorSubcoreMesh(core_axis_name="c", subcore_axis_name="s")

    scratch = (
        [pltpu.VMEM((rows_per_w, IDX_ROW), jnp.int32)]
        + [pltpu.VMEM((C, D_EMBED), jnp.float32) for _ in range(NBUF)]
        + [pltpu.SemaphoreType.DMA for _ in range(2 * NBUF)]
    )

    @functools.partial(
        pl.kernel,
        mesh=mesh,
        out_type=jax.ShapeDtypeStruct((n_idx_rows * IDX_ROW, D_EMBED),
                                      jnp.float32),
        scratch_types=scratch,
        compiler_params=pltpu.CompilerParams(use_tc_tiling_on_sc=False),
    )
    def emb_kernel(table_hbm, idx_hbm, out_hbm, idx_all, *scr):
        row_bufs = scr[:NBUF]
        gsems = scr[NBUF:2 * NBUF]
        osems = scr[2 * NBUF:3 * NBUF]

        wid = lax.axis_index("s") * _NC + lax.axis_index("c")
        row0 = wid * rows_per_w

        pltpu.sync_copy(idx_hbm.at[pl.ds(row0, rows_per_w)], idx_all)

        def start_group(g, b):
            for j in range(K):
                pltpu.async_copy(
                    table_hbm.at[idx_all.at[g * K + j]],
                    row_bufs[b].at[pl.ds(j * IDX_ROW, IDX_ROW)],
                    gsems[b],
                )

        def drain_gather(b):
            pltpu.make_async_copy(
                table_hbm.at[pl.ds(0, C)], row_bufs[b], gsems[b]).wait()

        def drain_out(b):
            pltpu.make_async_copy(
                row_bufs[b], out_hbm.at[pl.ds(0, C)], osems[b]).wait()

        def scale(b):
            rows_v = row_bufs[b]

            @plsc.parallel_loop(0, C, unroll=UNROLL)
            def _(i):
                for j in range(D_EMBED // _L):
                    sl = pl.ds(j * _L, _L)
                    rows_v[i, sl] = rows_v[i, sl] * SCALE

        start_group(0, 0)

        def outer(i0, carry):
            for b in range(NBUF):
                g = i0 * NBUF + b
                nb = (b + 1) % NBUF
                nxt = g + 1

                @pl.when(nxt < n_groups)
                def _():
                    @pl.when(g >= NBUF - 1)
                    def _():
                        drain_out(nb)

                    start_group(nxt, nb)

                drain_gather(b)
                scale(b)
                pltpu.async_copy(
                    row_bufs[b],
                    out_hbm.at[pl.ds((row0 + g * K) * IDX_ROW, C)],
                    osems[b],
                )
            return carry

        lax.fori_loop(0, n_groups // NBUF, outer, 0)

        for g in range(n_groups - NBUF, n_groups):
            drain_out(g % NBUF)

    return emb_kernel


def kernel(x, emb_tab):
    b0, b1 = x.shape
    n = b0 * b1
    idx2 = x.astype(jnp.int32).reshape(n // IDX_ROW, IDX_ROW)
    out = _make_kernel(n // IDX_ROW)(emb_tab, idx2)
    return out.reshape(b0, b1, D_EMBED)
